# baseline (speedup 1.0000x reference)
import functools

import jax
import jax.numpy as jnp
from jax import lax
from jax.experimental import pallas as pl
from jax.experimental.pallas import tpu as pltpu
from jax.experimental.pallas import tpu_sc as plsc

NC = 2
NS = 16
CE = 128
SB = 8
NHS = 1
CORE0_SHARE_NUM = 2
CORE0_SHARE_DEN = 3


def _wr_body(coeff_ref, basesf_ref, out_ref):
    out_ref[...] = jnp.dot(coeff_ref[...], basesf_ref[...],
                           preferred_element_type=jnp.float32)


def _compute_w2(coeff, bases):
    r, b = coeff.shape
    h = bases.shape[1]
    basesf = bases.reshape(b, h * h)
    wrf = pl.pallas_call(
        _wr_body,
        out_shape=jax.ShapeDtypeStruct((r, h * h), jnp.float32),
    )(coeff, basesf)
    wr = wrf.reshape(r, h, h)
    return jnp.concatenate([wr[0::2], wr[1::2]], axis=2)


def _xr_body(x_ref, w2_ref, out_ref, *, nr):
    x = x_ref[...]
    h = x.shape[1]
    for r2 in range(nr // 2):
        y = jnp.dot(x, w2_ref[r2], preferred_element_type=jnp.float32)
        out_ref[:, 2 * r2, :] = y[:, :h]
        out_ref[:, 2 * r2 + 1, :] = y[:, h:]


def _compute_xr(x, w2, bn=400):
    n, h = x.shape
    nr = w2.shape[0] * 2
    return pl.pallas_call(
        functools.partial(_xr_body, nr=nr),
        grid=(n // bn,),
        in_specs=[
            pl.BlockSpec((bn, h), lambda i: (i, 0)),
            pl.BlockSpec((nr // 2, h, 2 * h), lambda i: (0, 0, 0)),
        ],
        out_specs=pl.BlockSpec((bn, nr, h), lambda i: (i, 0, 0)),
        out_shape=jax.ShapeDtypeStruct((n, nr, h), jnp.float32),
    )(x, w2)


def _combine_body(p_ref, x_ref, lw_ref, b_ref, out_ref, *, relu):
    y = (p_ref[0] + p_ref[1]
         + jnp.dot(x_ref[...], lw_ref[...], preferred_element_type=jnp.float32)
         + b_ref[...])
    if relu:
        y = jnp.maximum(y, 0.0)
    out_ref[...] = y


def _combine(partials, x, loop_w, bias, relu, bn=400):
    n, h = x.shape
    return pl.pallas_call(
        functools.partial(_combine_body, relu=relu),
        grid=(n // bn,),
        in_specs=[
            pl.BlockSpec((2, bn, h), lambda i: (0, i, 0)),
            pl.BlockSpec((bn, h), lambda i: (i, 0)),
            pl.BlockSpec((h, h), lambda i: (0, 0)),
            pl.BlockSpec((1, h), lambda i: (0, 0)),
        ],
        out_specs=pl.BlockSpec((bn, h), lambda i: (i, 0)),
        out_shape=jax.ShapeDtypeStruct((n, h), jnp.float32),
    )(partials, x, loop_w, bias.reshape(1, h))


def _make_sc_scatter(n, h, r, ep):
    kw = ep // (NC * NS * CE)
    rpt = (n // NS) & ~7
    tail = n - rpt * NS
    mesh = plsc.VectorSubcoreMesh(core_axis_name="c", subcore_axis_name="s",
                                  num_cores=NC, num_subcores=NS)

    @functools.partial(
        pl.kernel,
        mesh=mesh,
        out_type=jax.ShapeDtypeStruct((NC, n, h), jnp.float32),
        scratch_types=[
            pltpu.VMEM((CE,), jnp.int32),
            pltpu.VMEM((CE,), jnp.int32),
            pltpu.VMEM((CE,), jnp.int32),
            pltpu.VMEM((CE,), jnp.int32),
            pltpu.VMEM((CE, 16), jnp.float32),
            pltpu.VMEM((CE, h), jnp.float32),
            pltpu.VMEM_SHARED((n, h), jnp.float32),
            pltpu.SemaphoreType.DMA,
        ],
    )
    def sc_scatter(xr_hbm, src_hbm, ety_hbm, dst_hbm, norm_hbm, out_hbm,
                   src_v, ety_v, idx_v, dst_v, norm_v, rows_v, acc, sem):
        c = lax.axis_index("c")
        s = lax.axis_index("s")
        wid = s * NC + c

        z16 = jnp.zeros((16,), jnp.float32)

        def zrow(e, carry):
            for hh in range(h // 16):
                rows_v[e, pl.ds(hh * 16, 16)] = z16
            return carry

        lax.fori_loop(0, CE, zrow, 0)
        full, rem = rpt // CE, rpt % CE
        for q in range(full):
            pltpu.sync_copy(rows_v, acc.at[pl.ds(s * rpt + q * CE, CE)])
        if rem:
            pltpu.sync_copy(rows_v.at[pl.ds(0, rem)],
                            acc.at[pl.ds(s * rpt + full * CE, rem)])
        if tail:
            @pl.when(s == NS - 1)
            def _():
                pltpu.sync_copy(rows_v.at[pl.ds(0, tail)],
                                acc.at[pl.ds(NS * rpt, tail)])
        plsc.subcore_barrier()

        k0 = (CORE0_SHARE_NUM * 2 * kw) // CORE0_SHARE_DEN
        k1 = 2 * kw - k0
        kcnt = jnp.where(c == 0, k0, k1)
        cbase = jnp.where(c == 0, s * k0, NS * k0 + s * k1)

        def chunk(k, carry):
            base = (cbase + k) * CE
            pltpu.sync_copy(src_hbm.at[pl.ds(base, CE)], src_v)
            pltpu.sync_copy(ety_hbm.at[pl.ds(base, CE)], ety_v)
            pltpu.sync_copy(dst_hbm.at[pl.ds(base, CE)], dst_v)
            pltpu.sync_copy(norm_hbm.at[pl.ds(base, CE)], norm_v)

            def mkidx(i, cc):
                sl = pl.ds(i * 16, 16)
                idx_v[sl] = src_v[sl] * r + ety_v[sl]
                return cc

            lax.fori_loop(0, CE // 16, mkidx, 0)
            pltpu.async_copy(xr_hbm.at[idx_v], rows_v, sem).wait()

            def scale(e, cc):
                nb = norm_v[e]
                for hh in range(h // 16):
                    sl = pl.ds(hh * 16, 16)
                    rows_v[e, sl] = rows_v[e, sl] * nb
                return cc

            lax.fori_loop(0, CE, scale, 0)
            pltpu.sync_copy(rows_v, acc.at[dst_v], add=True)
            return carry

        lax.fori_loop(0, kcnt, chunk, 0)
        plsc.subcore_barrier()
        pltpu.sync_copy(acc.at[pl.ds(s * rpt, rpt)],
                        out_hbm.at[c, pl.ds(s * rpt, rpt)])
        if tail:
            @pl.when(s == NS - 1)
            def _():
                pltpu.sync_copy(acc.at[pl.ds(NS * rpt, tail)],
                                out_hbm.at[c, pl.ds(NS * rpt, tail)])

    return sc_scatter


def _layer(x, srcp, etyp, dstp, normp, bases, coeff, loop_w, bias, relu):
    n, h = x.shape
    r = coeff.shape[0]
    ep = srcp.shape[0]
    w2 = _compute_w2(coeff, bases)
    xr = _compute_xr(x, w2)
    xr_flat = xr.reshape(n * r, h)
    sc = _make_sc_scatter(n, h, r, ep)
    partials = sc(xr_flat, srcp, etyp, dstp, normp)
    return _combine(partials, x, loop_w, bias, relu)


def kernel(nids, edge_index, etypes, norm, edge_weights, emb,
           bases1, coeff1, loop1, bias1,
           bases2, coeff2, loop2, bias2):
    x = emb
    e = etypes.shape[0]
    epw = NC * NS * CE
    ep = ((e + epw - 1) // epw) * epw
    pad = ep - e
    srcp = jnp.pad(edge_index[0], (0, pad))
    dstp = jnp.pad(edge_index[1], (0, pad))
    etyp = jnp.pad(etypes, (0, pad))
    normp = jnp.broadcast_to(jnp.pad(norm[:, 0], (0, pad))[:, None],
                             (ep, 16))

    hmid = _layer(x, srcp, etyp, dstp, normp,
                  bases1, coeff1, loop1, bias1, relu=True)
    out = _layer(hmid, srcp, etyp, dstp, normp,
                 bases2, coeff2, loop2, bias2, relu=False)
    return out


# speedup vs baseline: 1.0501x; 1.0501x over previous
import functools

import jax
import jax.numpy as jnp
from jax import lax
from jax.experimental import pallas as pl
from jax.experimental.pallas import tpu as pltpu
from jax.experimental.pallas import tpu_sc as plsc

NC = 2
NS = 16
CE = 128
SB = 8
NHS = 1
CORE0_SHARE_NUM = 4
CORE0_SHARE_DEN = 7


def _wr_body(coeff_ref, basesf_ref, out_ref):
    out_ref[...] = jnp.dot(coeff_ref[...], basesf_ref[...],
                           preferred_element_type=jnp.float32)


def _compute_w2(coeff, bases):
    r, b = coeff.shape
    h = bases.shape[1]
    basesf = bases.reshape(b, h * h)
    wrf = pl.pallas_call(
        _wr_body,
        out_shape=jax.ShapeDtypeStruct((r, h * h), jnp.float32),
    )(coeff, basesf)
    wr = wrf.reshape(r, h, h)
    return jnp.concatenate([wr[0::2], wr[1::2]], axis=2)


def _xr_body(x_ref, w2_ref, out_ref, *, nr):
    x = x_ref[...]
    h = x.shape[1]
    for r2 in range(nr // 2):
        y = jnp.dot(x, w2_ref[r2], preferred_element_type=jnp.float32)
        out_ref[:, 2 * r2, :] = y[:, :h]
        out_ref[:, 2 * r2 + 1, :] = y[:, h:]


def _compute_xr(x, w2, bn=400):
    n, h = x.shape
    nr = w2.shape[0] * 2
    return pl.pallas_call(
        functools.partial(_xr_body, nr=nr),
        grid=(n // bn,),
        in_specs=[
            pl.BlockSpec((bn, h), lambda i: (i, 0)),
            pl.BlockSpec((nr // 2, h, 2 * h), lambda i: (0, 0, 0)),
        ],
        out_specs=pl.BlockSpec((bn, nr, h), lambda i: (i, 0, 0)),
        out_shape=jax.ShapeDtypeStruct((n, nr, h), jnp.float32),
    )(x, w2)


def _combine_body(p_ref, x_ref, lw_ref, b_ref, out_ref, *, relu):
    y = (p_ref[0] + p_ref[1]
         + jnp.dot(x_ref[...], lw_ref[...], preferred_element_type=jnp.float32)
         + b_ref[...])
    if relu:
        y = jnp.maximum(y, 0.0)
    out_ref[...] = y


def _combine(partials, x, loop_w, bias, relu, bn=400):
    n, h = x.shape
    return pl.pallas_call(
        functools.partial(_combine_body, relu=relu),
        grid=(n // bn,),
        in_specs=[
            pl.BlockSpec((2, bn, h), lambda i: (0, i, 0)),
            pl.BlockSpec((bn, h), lambda i: (i, 0)),
            pl.BlockSpec((h, h), lambda i: (0, 0)),
            pl.BlockSpec((1, h), lambda i: (0, 0)),
        ],
        out_specs=pl.BlockSpec((bn, h), lambda i: (i, 0)),
        out_shape=jax.ShapeDtypeStruct((n, h), jnp.float32),
    )(partials, x, loop_w, bias.reshape(1, h))


def _make_sc_scatter(n, h, r, ep):
    kw = ep // (NC * NS * CE)
    rpt = (n // NS) & ~7
    tail = n - rpt * NS
    mesh = plsc.VectorSubcoreMesh(core_axis_name="c", subcore_axis_name="s",
                                  num_cores=NC, num_subcores=NS)

    @functools.partial(
        pl.kernel,
        mesh=mesh,
        out_type=jax.ShapeDtypeStruct((NC, n, h), jnp.float32),
        scratch_types=[
            pltpu.VMEM((CE,), jnp.int32),
            pltpu.VMEM((CE,), jnp.int32),
            pltpu.VMEM((CE,), jnp.int32),
            pltpu.VMEM((CE,), jnp.int32),
            pltpu.VMEM((CE, 16), jnp.float32),
            pltpu.VMEM((CE, h), jnp.float32),
            pltpu.VMEM_SHARED((n, h), jnp.float32),
            pltpu.SemaphoreType.DMA,
        ],
    )
    def sc_scatter(xr_hbm, src_hbm, ety_hbm, dst_hbm, norm_hbm, out_hbm,
                   src_v, ety_v, idx_v, dst_v, norm_v, rows_v, acc, sem):
        c = lax.axis_index("c")
        s = lax.axis_index("s")
        wid = s * NC + c

        z16 = jnp.zeros((16,), jnp.float32)

        def zrow(e, carry):
            for hh in range(h // 16):
                rows_v[e, pl.ds(hh * 16, 16)] = z16
            return carry

        lax.fori_loop(0, CE, zrow, 0)
        full, rem = rpt // CE, rpt % CE
        for q in range(full):
            pltpu.sync_copy(rows_v, acc.at[pl.ds(s * rpt + q * CE, CE)])
        if rem:
            pltpu.sync_copy(rows_v.at[pl.ds(0, rem)],
                            acc.at[pl.ds(s * rpt + full * CE, rem)])
        if tail:
            @pl.when(s == NS - 1)
            def _():
                pltpu.sync_copy(rows_v.at[pl.ds(0, tail)],
                                acc.at[pl.ds(NS * rpt, tail)])
        plsc.subcore_barrier()

        k0 = (CORE0_SHARE_NUM * 2 * kw) // CORE0_SHARE_DEN
        k1 = 2 * kw - k0
        kcnt = jnp.where(c == 0, k0, k1)
        cbase = jnp.where(c == 0, s * k0, NS * k0 + s * k1)

        def chunk(k, carry):
            base = (cbase + k) * CE
            pltpu.sync_copy(src_hbm.at[pl.ds(base, CE)], src_v)
            pltpu.sync_copy(ety_hbm.at[pl.ds(base, CE)], ety_v)
            pltpu.sync_copy(dst_hbm.at[pl.ds(base, CE)], dst_v)
            pltpu.sync_copy(norm_hbm.at[pl.ds(base, CE)], norm_v)

            def mkidx(i, cc):
                sl = pl.ds(i * 16, 16)
                idx_v[sl] = src_v[sl] * r + ety_v[sl]
                return cc

            lax.fori_loop(0, CE // 16, mkidx, 0)
            pltpu.async_copy(xr_hbm.at[idx_v], rows_v, sem).wait()

            def scale(e, cc):
                nb = norm_v[e]
                for hh in range(h // 16):
                    sl = pl.ds(hh * 16, 16)
                    rows_v[e, sl] = rows_v[e, sl] * nb
                return cc

            lax.fori_loop(0, CE, scale, 0)
            pltpu.sync_copy(rows_v, acc.at[dst_v], add=True)
            return carry

        lax.fori_loop(0, kcnt, chunk, 0)
        plsc.subcore_barrier()
        pltpu.sync_copy(acc.at[pl.ds(s * rpt, rpt)],
                        out_hbm.at[c, pl.ds(s * rpt, rpt)])
        if tail:
            @pl.when(s == NS - 1)
            def _():
                pltpu.sync_copy(acc.at[pl.ds(NS * rpt, tail)],
                                out_hbm.at[c, pl.ds(NS * rpt, tail)])

    return sc_scatter


def _layer(x, srcp, etyp, dstp, normp, bases, coeff, loop_w, bias, relu):
    n, h = x.shape
    r = coeff.shape[0]
    ep = srcp.shape[0]
    w2 = _compute_w2(coeff, bases)
    xr = _compute_xr(x, w2)
    xr_flat = xr.reshape(n * r, h)
    sc = _make_sc_scatter(n, h, r, ep)
    partials = sc(xr_flat, srcp, etyp, dstp, normp)
    return _combine(partials, x, loop_w, bias, relu)


def kernel(nids, edge_index, etypes, norm, edge_weights, emb,
           bases1, coeff1, loop1, bias1,
           bases2, coeff2, loop2, bias2):
    x = emb
    e = etypes.shape[0]
    epw = NC * NS * CE
    ep = ((e + epw - 1) // epw) * epw
    pad = ep - e
    srcp = jnp.pad(edge_index[0], (0, pad))
    dstp = jnp.pad(edge_index[1], (0, pad))
    etyp = jnp.pad(etypes, (0, pad))
    normp = jnp.broadcast_to(jnp.pad(norm[:, 0], (0, pad))[:, None],
                             (ep, 16))

    hmid = _layer(x, srcp, etyp, dstp, normp,
                  bases1, coeff1, loop1, bias1, relu=True)
    out = _layer(hmid, srcp, etyp, dstp, normp,
                 bases2, coeff2, loop2, bias2, relu=False)
    return out


# speedup vs baseline: 1.0533x; 1.0030x over previous
import functools

import jax
import jax.numpy as jnp
from jax import lax
from jax.experimental import pallas as pl
from jax.experimental.pallas import tpu as pltpu
from jax.experimental.pallas import tpu_sc as plsc

NC = 2
NS = 16
CE = 128
SB = 8
NHS = 1
CORE0_SHARE_NUM = 3
CORE0_SHARE_DEN = 5


def _wr_body(coeff_ref, basesf_ref, out_ref):
    out_ref[...] = jnp.dot(coeff_ref[...], basesf_ref[...],
                           preferred_element_type=jnp.float32)


def _compute_w2(coeff, bases):
    r, b = coeff.shape
    h = bases.shape[1]
    basesf = bases.reshape(b, h * h)
    wrf = pl.pallas_call(
        _wr_body,
        out_shape=jax.ShapeDtypeStruct((r, h * h), jnp.float32),
    )(coeff, basesf)
    wr = wrf.reshape(r, h, h)
    return jnp.concatenate([wr[0::2], wr[1::2]], axis=2)


def _xr_body(x_ref, w2_ref, out_ref, *, nr):
    x = x_ref[...]
    h = x.shape[1]
    for r2 in range(nr // 2):
        y = jnp.dot(x, w2_ref[r2], preferred_element_type=jnp.float32)
        out_ref[:, 2 * r2, :] = y[:, :h]
        out_ref[:, 2 * r2 + 1, :] = y[:, h:]


def _compute_xr(x, w2, bn=400):
    n, h = x.shape
    nr = w2.shape[0] * 2
    return pl.pallas_call(
        functools.partial(_xr_body, nr=nr),
        grid=(n // bn,),
        in_specs=[
            pl.BlockSpec((bn, h), lambda i: (i, 0)),
            pl.BlockSpec((nr // 2, h, 2 * h), lambda i: (0, 0, 0)),
        ],
        out_specs=pl.BlockSpec((bn, nr, h), lambda i: (i, 0, 0)),
        out_shape=jax.ShapeDtypeStruct((n, nr, h), jnp.float32),
    )(x, w2)


def _combine_body(p_ref, x_ref, lw_ref, b_ref, out_ref, *, relu):
    y = (p_ref[0] + p_ref[1]
         + jnp.dot(x_ref[...], lw_ref[...], preferred_element_type=jnp.float32)
         + b_ref[...])
    if relu:
        y = jnp.maximum(y, 0.0)
    out_ref[...] = y


def _combine(partials, x, loop_w, bias, relu, bn=400):
    n, h = x.shape
    return pl.pallas_call(
        functools.partial(_combine_body, relu=relu),
        grid=(n // bn,),
        in_specs=[
            pl.BlockSpec((2, bn, h), lambda i: (0, i, 0)),
            pl.BlockSpec((bn, h), lambda i: (i, 0)),
            pl.BlockSpec((h, h), lambda i: (0, 0)),
            pl.BlockSpec((1, h), lambda i: (0, 0)),
        ],
        out_specs=pl.BlockSpec((bn, h), lambda i: (i, 0)),
        out_shape=jax.ShapeDtypeStruct((n, h), jnp.float32),
    )(partials, x, loop_w, bias.reshape(1, h))


def _make_sc_scatter(n, h, r, ep):
    kw = ep // (NC * NS * CE)
    rpt = (n // NS) & ~7
    tail = n - rpt * NS
    mesh = plsc.VectorSubcoreMesh(core_axis_name="c", subcore_axis_name="s",
                                  num_cores=NC, num_subcores=NS)

    @functools.partial(
        pl.kernel,
        mesh=mesh,
        out_type=jax.ShapeDtypeStruct((NC, n, h), jnp.float32),
        scratch_types=[
            pltpu.VMEM((CE,), jnp.int32),
            pltpu.VMEM((CE,), jnp.int32),
            pltpu.VMEM((CE,), jnp.int32),
            pltpu.VMEM((CE,), jnp.int32),
            pltpu.VMEM((CE, 16), jnp.float32),
            pltpu.VMEM((CE, h), jnp.float32),
            pltpu.VMEM_SHARED((n, h), jnp.float32),
            pltpu.SemaphoreType.DMA,
        ],
    )
    def sc_scatter(xr_hbm, src_hbm, ety_hbm, dst_hbm, norm_hbm, out_hbm,
                   src_v, ety_v, idx_v, dst_v, norm_v, rows_v, acc, sem):
        c = lax.axis_index("c")
        s = lax.axis_index("s")
        wid = s * NC + c

        z16 = jnp.zeros((16,), jnp.float32)

        def zrow(e, carry):
            for hh in range(h // 16):
                rows_v[e, pl.ds(hh * 16, 16)] = z16
            return carry

        lax.fori_loop(0, CE, zrow, 0)
        full, rem = rpt // CE, rpt % CE
        for q in range(full):
            pltpu.sync_copy(rows_v, acc.at[pl.ds(s * rpt + q * CE, CE)])
        if rem:
            pltpu.sync_copy(rows_v.at[pl.ds(0, rem)],
                            acc.at[pl.ds(s * rpt + full * CE, rem)])
        if tail:
            @pl.when(s == NS - 1)
            def _():
                pltpu.sync_copy(rows_v.at[pl.ds(0, tail)],
                                acc.at[pl.ds(NS * rpt, tail)])
        plsc.subcore_barrier()

        k0 = (CORE0_SHARE_NUM * 2 * kw) // CORE0_SHARE_DEN
        k1 = 2 * kw - k0
        kcnt = jnp.where(c == 0, k0, k1)
        cbase = jnp.where(c == 0, s * k0, NS * k0 + s * k1)

        def chunk(k, carry):
            base = (cbase + k) * CE
            pltpu.sync_copy(src_hbm.at[pl.ds(base, CE)], src_v)
            pltpu.sync_copy(ety_hbm.at[pl.ds(base, CE)], ety_v)
            pltpu.sync_copy(dst_hbm.at[pl.ds(base, CE)], dst_v)
            pltpu.sync_copy(norm_hbm.at[pl.ds(base, CE)], norm_v)

            def mkidx(i, cc):
                sl = pl.ds(i * 16, 16)
                idx_v[sl] = src_v[sl] * r + ety_v[sl]
                return cc

            lax.fori_loop(0, CE // 16, mkidx, 0)
            pltpu.async_copy(xr_hbm.at[idx_v], rows_v, sem).wait()

            def scale(e, cc):
                nb = norm_v[e]
                for hh in range(h // 16):
                    sl = pl.ds(hh * 16, 16)
                    rows_v[e, sl] = rows_v[e, sl] * nb
                return cc

            lax.fori_loop(0, CE, scale, 0)
            pltpu.sync_copy(rows_v, acc.at[dst_v], add=True)
            return carry

        lax.fori_loop(0, kcnt, chunk, 0)
        plsc.subcore_barrier()
        pltpu.sync_copy(acc.at[pl.ds(s * rpt, rpt)],
                        out_hbm.at[c, pl.ds(s * rpt, rpt)])
        if tail:
            @pl.when(s == NS - 1)
            def _():
                pltpu.sync_copy(acc.at[pl.ds(NS * rpt, tail)],
                                out_hbm.at[c, pl.ds(NS * rpt, tail)])

    return sc_scatter


def _layer(x, srcp, etyp, dstp, normp, bases, coeff, loop_w, bias, relu):
    n, h = x.shape
    r = coeff.shape[0]
    ep = srcp.shape[0]
    w2 = _compute_w2(coeff, bases)
    xr = _compute_xr(x, w2)
    xr_flat = xr.reshape(n * r, h)
    sc = _make_sc_scatter(n, h, r, ep)
    partials = sc(xr_flat, srcp, etyp, dstp, normp)
    return _combine(partials, x, loop_w, bias, relu)


def kernel(nids, edge_index, etypes, norm, edge_weights, emb,
           bases1, coeff1, loop1, bias1,
           bases2, coeff2, loop2, bias2):
    x = emb
    e = etypes.shape[0]
    epw = NC * NS * CE
    ep = ((e + epw - 1) // epw) * epw
    pad = ep - e
    srcp = jnp.pad(edge_index[0], (0, pad))
    dstp = jnp.pad(edge_index[1], (0, pad))
    etyp = jnp.pad(etypes, (0, pad))
    normp = jnp.broadcast_to(jnp.pad(norm[:, 0], (0, pad))[:, None],
                             (ep, 16))

    hmid = _layer(x, srcp, etyp, dstp, normp,
                  bases1, coeff1, loop1, bias1, relu=True)
    out = _layer(hmid, srcp, etyp, dstp, normp,
                 bases2, coeff2, loop2, bias2, relu=False)
    return out
